# Initial kernel scaffold; baseline (speedup 1.0000x reference)
import jax
import jax.numpy as jnp
from jax.experimental import pallas as pl


def kernel(x, edge_index, W1, b1, W2, b2, Wq, bq, Wk, bk, Wv, bv):
    raise NotImplementedError("write your pallas kernel here")


# speedup vs baseline: 5.8581x; 5.8581x over previous
import functools

import jax
import jax.numpy as jnp
from jax import lax
from jax.experimental import pallas as pl
from jax.experimental.pallas import tpu as pltpu
from jax.experimental.pallas import tpu_sc as plsc

N = 100000
D = 64
NC = 2
NS = 16
NW = NC * NS
EPT = 40960
EP = NW * EPT
BLK = 4096
G = 128
CHUNK = 12500
NK = 4
ACC_ROWS = 12544
ZROWS = 112
RB = 4000

_mesh = plsc.VectorSubcoreMesh(
    core_axis_name="c", subcore_axis_name="s", num_cores=NC, num_subcores=NS)
_sc_params = pltpu.CompilerParams(use_tc_tiling_on_sc=False,
                                  needs_layout_passes=False)


@functools.partial(
    pl.kernel,
    out_type=jax.ShapeDtypeStruct((NW, N), jnp.float32),
    mesh=_mesh,
    compiler_params=_sc_params,
    scratch_types=[
        pltpu.VMEM((N,), jnp.float32),
        pltpu.VMEM((BLK,), jnp.int32),
    ],
)
def _deg_kernel(dst_hbm, out_hbm, deg_local, dst_blk):
    cid = lax.axis_index("c")
    sid = lax.axis_index("s")
    wid = cid * NS + sid

    zf = jnp.zeros((16,), jnp.float32)

    def zero_body(i, _):
        deg_local[pl.ds(i * 16, 16)] = zf
        return 0

    lax.fori_loop(0, N // 16, zero_body, 0)

    ones = jnp.ones((16,), jnp.float32)

    def blk_body(b, _):
        off = wid * EPT + b * BLK
        pltpu.sync_copy(dst_hbm.at[pl.ds(off, BLK)], dst_blk)

        def batch_body(i, _):
            d = dst_blk[pl.ds(i * 16, 16)]
            plsc.addupdate_scatter(deg_local, [d], ones, mask=d < N)
            return 0

        lax.fori_loop(0, BLK // 16, batch_body, 0)
        return 0

    lax.fori_loop(0, EPT // BLK, blk_body, 0)
    pltpu.sync_copy(deg_local, out_hbm.at[wid])


@functools.partial(
    pl.kernel,
    out_type=jax.ShapeDtypeStruct((N, D), jnp.float32),
    mesh=_mesh,
    compiler_params=_sc_params,
    scratch_types=[
        pltpu.VMEM_SHARED((ACC_ROWS, D), jnp.float32),
        pltpu.VMEM((BLK,), jnp.int32),
        pltpu.VMEM((BLK,), jnp.int32),
        pltpu.VMEM((BLK + G,), jnp.int32),
        pltpu.VMEM((BLK + G,), jnp.int32),
        pltpu.VMEM((1, G), jnp.int32),
        pltpu.VMEM((G, D), jnp.float32),
        pltpu.VMEM((ZROWS, D), jnp.float32),
        pltpu.SemaphoreType.DMA,
    ],
)
def _segsum_kernel(g_hbm, src_hbm, dst_hbm, out_hbm,
                   acc, src_blk, dst_blk, sel_src, sel_dst,
                   stage_idx, rows, zero_buf, sem):
    cid = lax.axis_index("c")
    sid = lax.axis_index("s")
    wid = cid * NS + sid

    zf = jnp.zeros((16,), jnp.float32)

    def zb_body(i, _):
        zero_buf[i // 4, pl.ds((i % 4) * 16, 16)] = zf
        return 0

    lax.fori_loop(0, ZROWS * (D // 16), zb_body, 0)

    trash16 = jnp.full((16,), CHUNK, jnp.int32)
    zeros16 = jnp.zeros((16,), jnp.int32)

    for k in range(NK):
        lo = (NK * cid + k) * CHUNK

        for z in range(ACC_ROWS // NS // ZROWS):
            pltpu.sync_copy(
                zero_buf,
                acc.at[pl.ds(sid * (ACC_ROWS // NS) + z * ZROWS, ZROWS)])
        plsc.subcore_barrier()

        def blk_body(b, _, lo=lo):
            off = sid * (EP // NS) + b * BLK
            pltpu.sync_copy(src_hbm.at[pl.ds(off, BLK)], src_blk)
            pltpu.sync_copy(dst_hbm.at[pl.ds(off, BLK)], dst_blk)

            def compact_body(i, cnt, lo=lo):
                d = dst_blk[pl.ds(i * 16, 16)]
                sv = src_blk[pl.ds(i * 16, 16)]
                m = (d >= lo) & (d < lo + CHUNK)
                plsc.store_compressed(sel_src.at[pl.ds(cnt, 16)], sv, mask=m)
                plsc.store_compressed(sel_dst.at[pl.ds(cnt, 16)], d - lo,
                                      mask=m)
                return cnt + jnp.sum(m.astype(jnp.int32))

            cnt = lax.fori_loop(0, BLK // 16, compact_body, 0)

            for p in range(G // 16):
                sel_dst[pl.ds(cnt + p * 16, 16)] = trash16
                sel_src[pl.ds(cnt + p * 16, 16)] = zeros16

            def group_body(j, _):
                pltpu.async_copy(
                    g_hbm.at[sel_src.at[pl.ds(j * G, G)]], rows, sem).wait()
                for p in range(G // 16):
                    stage_idx[0, pl.ds(p * 16, 16)] = (
                        sel_dst[pl.ds(j * G + p * 16, 16)])
                pltpu.sync_copy(rows, acc.at[stage_idx.at[0]], add=True)
                return 0

            lax.fori_loop(0, (cnt + G - 1) // G, group_body, 0)
            return 0

        lax.fori_loop(0, (EP // NS) // BLK, blk_body, 0)
        plsc.subcore_barrier()

        @pl.when(sid < 4)
        def _copy_lo(lo=lo):
            off = sid * 782
            pltpu.sync_copy(acc.at[pl.ds(off, 782)],
                            out_hbm.at[pl.ds(lo + off, 782)])

        @pl.when(sid >= 4)
        def _copy_hi(lo=lo):
            off = 4 * 782 + (sid - 4) * 781
            pltpu.sync_copy(acc.at[pl.ds(off, 781)],
                            out_hbm.at[pl.ds(lo + off, 781)])

        plsc.subcore_barrier()


def _dinv_of(degp_ref):
    deg = jnp.sum(degp_ref[0], axis=0) + 1.0
    return lax.rsqrt(deg)


def _dense_a_body(degp_ref, x_ref, w1_ref, g1_ref):
    dinv = _dinv_of(degp_ref)
    m = jnp.dot(x_ref[...], w1_ref[...], preferred_element_type=jnp.float32)
    g1_ref[...] = m * dinv[:, None]


def _dense_b_body(degp_ref, s1_ref, g1_ref, b1_ref, w2_ref, g2_ref):
    dinv = _dinv_of(degp_ref)
    h = jnp.maximum(dinv[:, None] * (s1_ref[...] + g1_ref[...])
                    + b1_ref[...], 0.0)
    m = jnp.dot(h, w2_ref[...], preferred_element_type=jnp.float32)
    g2_ref[...] = m * dinv[:, None]


def _dense_c_body(degp_ref, s2_ref, g2_ref, b2_ref, wv_ref, bv_ref, out_ref):
    dinv = _dinv_of(degp_ref)
    h = jnp.maximum(dinv[:, None] * (s2_ref[...] + g2_ref[...])
                    + b2_ref[...], 0.0)
    out_ref[...] = jnp.dot(h, wv_ref[...],
                           preferred_element_type=jnp.float32) + bv_ref[...]


def _row_spec():
    return pl.BlockSpec((RB, D), lambda i: (i, 0))


def _deg_spec():
    return pl.BlockSpec((1, NW, RB), lambda i: (i, 0, 0))


def _full_spec(r):
    return pl.BlockSpec((r, D), lambda i: (0, 0))


_dense_a = pl.pallas_call(
    _dense_a_body,
    grid=(N // RB,),
    in_specs=[_deg_spec(), _row_spec(), _full_spec(D)],
    out_specs=_row_spec(),
    out_shape=jax.ShapeDtypeStruct((N, D), jnp.float32),
)

_dense_b = pl.pallas_call(
    _dense_b_body,
    grid=(N // RB,),
    in_specs=[_deg_spec(), _row_spec(), _row_spec(), _full_spec(1),
              _full_spec(D)],
    out_specs=_row_spec(),
    out_shape=jax.ShapeDtypeStruct((N, D), jnp.float32),
)

_dense_c = pl.pallas_call(
    _dense_c_body,
    grid=(N // RB,),
    in_specs=[_deg_spec(), _row_spec(), _row_spec(), _full_spec(1),
              _full_spec(D), _full_spec(1)],
    out_specs=_row_spec(),
    out_shape=jax.ShapeDtypeStruct((N, D), jnp.float32),
)


def kernel(x, edge_index, W1, b1, W2, b2, Wq, bq, Wk, bk, Wv, bv):
    del Wq, bq, Wk, bk
    src = edge_index[0]
    dst = edge_index[1]
    pad = EP - src.shape[0]
    src_p = jnp.concatenate([src, jnp.zeros((pad,), jnp.int32)])
    dst_p = jnp.concatenate([dst, jnp.full((pad,), N, jnp.int32)])

    degp = _deg_kernel(dst_p)
    degp = degp.reshape(NW, N // RB, RB).transpose(1, 0, 2)
    g1 = _dense_a(degp, x, W1)
    s1 = _segsum_kernel(g1, src_p, dst_p)
    g2 = _dense_b(degp, s1, g1, b1.reshape(1, D), W2)
    s2 = _segsum_kernel(g2, src_p, dst_p)
    out = _dense_c(degp, s2, g2, b2.reshape(1, D), Wv.astype(jnp.float32),
                   bv.reshape(1, D))
    return out[:, None, :]


# speedup vs baseline: 16.2375x; 2.7718x over previous
import functools

import jax
import jax.numpy as jnp
from jax import lax
from jax.experimental import pallas as pl
from jax.experimental.pallas import tpu as pltpu
from jax.experimental.pallas import tpu_sc as plsc

N = 100000
D = 64
NC = 2
NS = 16
NW = NC * NS
EPT = 40960
EP = NW * EPT
BLK = 2048
G = 128
CHUNK = 10000
NK = 5
ACC_ROWS = 10032
ZROWS = 57
RB = 4000

_mesh = plsc.VectorSubcoreMesh(
    core_axis_name="c", subcore_axis_name="s", num_cores=NC, num_subcores=NS)
_sc_params = pltpu.CompilerParams(use_tc_tiling_on_sc=False,
                                  needs_layout_passes=False)


@functools.partial(
    pl.kernel,
    out_type=jax.ShapeDtypeStruct((NW, N), jnp.float32),
    mesh=_mesh,
    compiler_params=_sc_params,
    scratch_types=[
        pltpu.VMEM((N,), jnp.float32),
        pltpu.VMEM((BLK,), jnp.int32),
    ],
)
def _deg_kernel(dst_hbm, out_hbm, deg_local, dst_blk):
    cid = lax.axis_index("c")
    sid = lax.axis_index("s")
    wid = cid * NS + sid

    zf = jnp.zeros((16,), jnp.float32)

    def zero_body(i, _):
        deg_local[pl.ds(i * 16, 16)] = zf
        return 0

    lax.fori_loop(0, N // 16, zero_body, 0)

    ones = jnp.ones((16,), jnp.float32)

    def blk_body(b, _):
        off = wid * EPT + b * BLK
        pltpu.sync_copy(dst_hbm.at[pl.ds(off, BLK)], dst_blk)

        def batch_body(i, _):
            d = dst_blk[pl.ds(i * 16, 16)]
            plsc.addupdate_scatter(deg_local, [d], ones, mask=d < N)
            return 0

        lax.fori_loop(0, BLK // 16, batch_body, 0)
        return 0

    lax.fori_loop(0, EPT // BLK, blk_body, 0)
    pltpu.sync_copy(deg_local, out_hbm.at[wid])


NWAVE = 4
SG = NWAVE * G
SELCAP = 2 * BLK


@functools.partial(
    pl.kernel,
    out_type=jax.ShapeDtypeStruct((N, D), jnp.float32),
    mesh=_mesh,
    compiler_params=_sc_params,
    scratch_types=[
        pltpu.VMEM_SHARED((ACC_ROWS, D), jnp.float32),
        pltpu.VMEM((BLK,), jnp.int32),
        pltpu.VMEM((BLK,), jnp.int32),
        pltpu.VMEM((SELCAP + G,), jnp.int32),
        pltpu.VMEM((SELCAP + G,), jnp.int32),
        pltpu.VMEM((NWAVE, G), jnp.int32),
        pltpu.VMEM((SG, D), jnp.float32),
        pltpu.VMEM((ZROWS, D), jnp.float32),
        pltpu.SemaphoreType.DMA,
        pltpu.SemaphoreType.DMA,
    ],
)
def _segsum_kernel(g_hbm, src_hbm, dst_hbm, out_hbm,
                   acc, src_blk, dst_blk, sel_src, sel_dst,
                   stage_idx, rows, zero_buf, gsem, ssem):
    cid = lax.axis_index("c")
    sid = lax.axis_index("s")

    zf = jnp.zeros((16,), jnp.float32)

    def zb_body(i, _):
        zero_buf[i // 4, pl.ds((i % 4) * 16, 16)] = zf
        return 0

    lax.fori_loop(0, ZROWS * (D // 16), zb_body, 0)

    trash16 = jnp.full((16,), CHUNK, jnp.int32)
    zeros16 = jnp.zeros((16,), jnp.int32)

    def do_wave(base, ng_pred):
        gd, sd = [], []
        for g in range(NWAVE):
            def fire_g(g=g):
                gd.append(pltpu.async_copy(
                    g_hbm.at[sel_src.at[pl.ds(base + g * G, G)]],
                    rows.at[pl.ds(g * G, G)], gsem))
            if ng_pred is None:
                fire_g()
            else:
                pl.when(g < ng_pred)(fire_g)
        for g in range(NWAVE):
            if ng_pred is None:
                gd[g].wait()
            else:
                pl.when(g < ng_pred)(lambda g=g: gd[g].wait())
        for g in range(NWAVE):
            def fire_s(g=g):
                for p in range(G // 16):
                    stage_idx[g, pl.ds(p * 16, 16)] = (
                        sel_dst[pl.ds(base + g * G + p * 16, 16)])
                sd.append(pltpu.async_copy(
                    rows.at[pl.ds(g * G, G)],
                    acc.at[stage_idx.at[g]], ssem, add=True))
            if ng_pred is None:
                fire_s()
            else:
                pl.when(g < ng_pred)(fire_s)
        for g in range(len(sd)):
            if ng_pred is None:
                sd[g].wait()
            else:
                pl.when(g < ng_pred)(lambda g=g: sd[g].wait())

    for k in range(NK):
        lo = (NK * cid + k) * CHUNK

        for z in range(ACC_ROWS // NS // ZROWS):
            pltpu.sync_copy(
                zero_buf,
                acc.at[pl.ds(sid * (ACC_ROWS // NS) + z * ZROWS, ZROWS)])
        plsc.subcore_barrier()

        def blk_body(b, wp, lo=lo):
            off = sid * (EP // NS) + b * BLK
            pltpu.sync_copy(src_hbm.at[pl.ds(off, BLK)], src_blk)
            pltpu.sync_copy(dst_hbm.at[pl.ds(off, BLK)], dst_blk)

            def compact_body(i, cnt, lo=lo):
                d = dst_blk[pl.ds(i * 16, 16)]
                sv = src_blk[pl.ds(i * 16, 16)]
                m = (d >= lo) & (d < lo + CHUNK)
                plsc.store_compressed(sel_src.at[pl.ds(cnt, 16)], sv, mask=m)
                plsc.store_compressed(sel_dst.at[pl.ds(cnt, 16)], d - lo,
                                      mask=m)
                return cnt + jnp.sum(m.astype(jnp.int32))

            wp = lax.fori_loop(0, BLK // 16, compact_body, wp)

            nsg = wp // SG

            def sg_body(s, _):
                do_wave(s * SG, None)
                return 0

            lax.fori_loop(0, nsg, sg_body, 0)

            rem = wp - nsg * SG

            def shift_body(i, _):
                sel_src[pl.ds(i * 16, 16)] = sel_src[
                    pl.ds(nsg * SG + i * 16, 16)]
                sel_dst[pl.ds(i * 16, 16)] = sel_dst[
                    pl.ds(nsg * SG + i * 16, 16)]
                return 0

            @pl.when(nsg > 0)
            def _shift():
                lax.fori_loop(0, (rem + 15) // 16, shift_body, 0)

            return rem

        wp = lax.fori_loop(0, (EP // NS) // BLK, blk_body, 0)

        for p in range(G // 16):
            sel_dst[pl.ds(wp + p * 16, 16)] = trash16
            sel_src[pl.ds(wp + p * 16, 16)] = zeros16
        do_wave(0, (wp + G - 1) // G)
        plsc.subcore_barrier()

        off = sid * 625
        pltpu.sync_copy(acc.at[pl.ds(off, 625)],
                        out_hbm.at[pl.ds(lo + off, 625)])

        plsc.subcore_barrier()


def _dinv_of(degp_ref):
    deg = jnp.sum(degp_ref[0], axis=0) + 1.0
    return lax.rsqrt(deg)


def _dense_a_body(degp_ref, x_ref, w1_ref, g1_ref):
    dinv = _dinv_of(degp_ref)
    m = jnp.dot(x_ref[...], w1_ref[...], preferred_element_type=jnp.float32)
    g1_ref[...] = m * dinv[:, None]


def _dense_b_body(degp_ref, s1_ref, g1_ref, b1_ref, w2_ref, g2_ref):
    dinv = _dinv_of(degp_ref)
    h = jnp.maximum(dinv[:, None] * (s1_ref[...] + g1_ref[...])
                    + b1_ref[...], 0.0)
    m = jnp.dot(h, w2_ref[...], preferred_element_type=jnp.float32)
    g2_ref[...] = m * dinv[:, None]


def _dense_c_body(degp_ref, s2_ref, g2_ref, b2_ref, wv_ref, bv_ref, out_ref):
    dinv = _dinv_of(degp_ref)
    h = jnp.maximum(dinv[:, None] * (s2_ref[...] + g2_ref[...])
                    + b2_ref[...], 0.0)
    out_ref[...] = jnp.dot(h, wv_ref[...],
                           preferred_element_type=jnp.float32) + bv_ref[...]


def _row_spec():
    return pl.BlockSpec((RB, D), lambda i: (i, 0))


def _deg_spec():
    return pl.BlockSpec((1, NW, RB), lambda i: (i, 0, 0))


def _full_spec(r):
    return pl.BlockSpec((r, D), lambda i: (0, 0))


_dense_a = pl.pallas_call(
    _dense_a_body,
    grid=(N // RB,),
    in_specs=[_deg_spec(), _row_spec(), _full_spec(D)],
    out_specs=_row_spec(),
    out_shape=jax.ShapeDtypeStruct((N, D), jnp.float32),
)

_dense_b = pl.pallas_call(
    _dense_b_body,
    grid=(N // RB,),
    in_specs=[_deg_spec(), _row_spec(), _row_spec(), _full_spec(1),
              _full_spec(D)],
    out_specs=_row_spec(),
    out_shape=jax.ShapeDtypeStruct((N, D), jnp.float32),
)

_dense_c = pl.pallas_call(
    _dense_c_body,
    grid=(N // RB,),
    in_specs=[_deg_spec(), _row_spec(), _row_spec(), _full_spec(1),
              _full_spec(D), _full_spec(1)],
    out_specs=_row_spec(),
    out_shape=jax.ShapeDtypeStruct((N, D), jnp.float32),
)


def kernel(x, edge_index, W1, b1, W2, b2, Wq, bq, Wk, bk, Wv, bv):
    del Wq, bq, Wk, bk
    src = edge_index[0]
    dst = edge_index[1]
    pad = EP - src.shape[0]
    src_p = jnp.concatenate([src, jnp.zeros((pad,), jnp.int32)])
    dst_p = jnp.concatenate([dst, jnp.full((pad,), N, jnp.int32)])

    degp = _deg_kernel(dst_p)
    degp = degp.reshape(NW, N // RB, RB).transpose(1, 0, 2)
    g1 = _dense_a(degp, x, W1)
    s1 = _segsum_kernel(g1, src_p, dst_p)
    g2 = _dense_b(degp, s1, g1, b1.reshape(1, D), W2)
    s2 = _segsum_kernel(g2, src_p, dst_p)
    out = _dense_c(degp, s2, g2, b2.reshape(1, D), Wv.astype(jnp.float32),
                   bv.reshape(1, D))
    return out[:, None, :]


# speedup vs baseline: 19.5311x; 1.2028x over previous
import functools

import jax
import jax.numpy as jnp
from jax import lax
from jax.experimental import pallas as pl
from jax.experimental.pallas import tpu as pltpu
from jax.experimental.pallas import tpu_sc as plsc

N = 100000
D = 64
NC = 2
NS = 16
NW = NC * NS
EPT = 40960
EP = NW * EPT
BLK = 1024
G = 128
CHUNK = 10000
NK = 5
ACC_ROWS = 10032
ZROWS = 57
RB = 4000

_mesh = plsc.VectorSubcoreMesh(
    core_axis_name="c", subcore_axis_name="s", num_cores=NC, num_subcores=NS)
_sc_params = pltpu.CompilerParams(use_tc_tiling_on_sc=False,
                                  needs_layout_passes=False)


@functools.partial(
    pl.kernel,
    out_type=jax.ShapeDtypeStruct((NW, N), jnp.float32),
    mesh=_mesh,
    compiler_params=_sc_params,
    scratch_types=[
        pltpu.VMEM((N,), jnp.float32),
        pltpu.VMEM((BLK,), jnp.int32),
    ],
)
def _deg_kernel(dst_hbm, out_hbm, deg_local, dst_blk):
    cid = lax.axis_index("c")
    sid = lax.axis_index("s")
    wid = cid * NS + sid

    zf = jnp.zeros((16,), jnp.float32)

    def zero_body(i, _):
        deg_local[pl.ds(i * 16, 16)] = zf
        return 0

    lax.fori_loop(0, N // 16, zero_body, 0)

    ones = jnp.ones((16,), jnp.float32)

    def blk_body(b, _):
        off = wid * EPT + b * BLK
        pltpu.sync_copy(dst_hbm.at[pl.ds(off, BLK)], dst_blk)

        def batch_body(i, _):
            d = dst_blk[pl.ds(i * 16, 16)]
            plsc.addupdate_scatter(deg_local, [d], ones, mask=d < N)
            return 0

        lax.fori_loop(0, BLK // 16, batch_body, 0)
        return 0

    lax.fori_loop(0, EPT // BLK, blk_body, 0)
    pltpu.sync_copy(deg_local, out_hbm.at[wid])


NWAVE = 4
SG = NWAVE * G
SELCAP = SG + BLK
NBLK = (EP // NS) // BLK


@functools.partial(
    pl.kernel,
    out_type=jax.ShapeDtypeStruct((N, D), jnp.float32),
    mesh=_mesh,
    compiler_params=_sc_params,
    scratch_types=[
        pltpu.VMEM_SHARED((ACC_ROWS, D), jnp.float32),
        pltpu.VMEM((2, BLK), jnp.int32),
        pltpu.VMEM((2, BLK), jnp.int32),
        pltpu.VMEM((SELCAP + G,), jnp.int32),
        pltpu.VMEM((SELCAP + G,), jnp.int32),
        pltpu.VMEM((NWAVE, G), jnp.int32),
        pltpu.VMEM((SG, D), jnp.float32),
        pltpu.VMEM((ZROWS, D), jnp.float32),
        pltpu.SemaphoreType.DMA,
        pltpu.SemaphoreType.DMA,
        pltpu.SemaphoreType.DMA,
    ],
)
def _segsum_kernel(g_hbm, src_hbm, dst_hbm, out_hbm,
                   acc, src_blk, dst_blk, sel_src, sel_dst,
                   stage_idx, rows, zero_buf, gsem, ssem, stg_sem):
    cid = lax.axis_index("c")
    sid = lax.axis_index("s")

    zf = jnp.zeros((16,), jnp.float32)

    def zb_body(i, _):
        zero_buf[i // 4, pl.ds((i % 4) * 16, 16)] = zf
        return 0

    lax.fori_loop(0, ZROWS * (D // 16), zb_body, 0)

    trash16 = jnp.full((16,), CHUNK, jnp.int32)
    zeros16 = jnp.zeros((16,), jnp.int32)

    def do_wave(base, ng_pred):
        gd, sd = [], []
        for g in range(NWAVE):
            def fire_g(g=g):
                gd.append(pltpu.async_copy(
                    g_hbm.at[sel_src.at[pl.ds(base + g * G, G)]],
                    rows.at[pl.ds(g * G, G)], gsem))
            if ng_pred is None:
                fire_g()
            else:
                pl.when(g < ng_pred)(fire_g)
        for g in range(NWAVE):
            if ng_pred is None:
                gd[g].wait()
            else:
                pl.when(g < ng_pred)(lambda g=g: gd[g].wait())
        for g in range(NWAVE):
            def fire_s(g=g):
                for p in range(G // 16):
                    stage_idx[g, pl.ds(p * 16, 16)] = (
                        sel_dst[pl.ds(base + g * G + p * 16, 16)])
                sd.append(pltpu.async_copy(
                    rows.at[pl.ds(g * G, G)],
                    acc.at[stage_idx.at[g]], ssem, add=True))
            if ng_pred is None:
                fire_s()
            else:
                pl.when(g < ng_pred)(fire_s)
        for g in range(len(sd)):
            if ng_pred is None:
                sd[g].wait()
            else:
                pl.when(g < ng_pred)(lambda g=g: sd[g].wait())

    for k in range(NK):
        lo = (NK * cid + k) * CHUNK

        for z in range(ACC_ROWS // NS // ZROWS):
            pltpu.sync_copy(
                zero_buf,
                acc.at[pl.ds(sid * (ACC_ROWS // NS) + z * ZROWS, ZROWS)])
        plsc.subcore_barrier()

        def stage_block(b):
            off = sid * (EP // NS) + b * BLK
            buf = lax.rem(b, 2)
            pltpu.async_copy(src_hbm.at[pl.ds(off, BLK)], src_blk.at[buf],
                             stg_sem)
            pltpu.async_copy(dst_hbm.at[pl.ds(off, BLK)], dst_blk.at[buf],
                             stg_sem)

        stage_block(0)

        def blk_body(b, wp, lo=lo):
            pltpu.make_async_copy(
                src_hbm.at[pl.ds(0, BLK)], src_blk.at[0], stg_sem).wait()
            pltpu.make_async_copy(
                src_hbm.at[pl.ds(0, BLK)], dst_blk.at[0], stg_sem).wait()
            bi = lax.rem(b, 2)

            @pl.when(b + 1 < NBLK)
            def _prefetch():
                stage_block(b + 1)

            def compact_body(i, cnt, lo=lo, bi=bi):
                d = dst_blk[bi, pl.ds(i * 16, 16)]
                sv = src_blk[bi, pl.ds(i * 16, 16)]
                m = (d >= lo) & (d < lo + CHUNK)
                plsc.store_compressed(sel_src.at[pl.ds(cnt, 16)], sv, mask=m)
                plsc.store_compressed(sel_dst.at[pl.ds(cnt, 16)], d - lo,
                                      mask=m)
                return cnt + jnp.sum(m.astype(jnp.int32))

            wp = lax.fori_loop(0, BLK // 16, compact_body, wp)

            nsg = wp // SG

            def sg_body(s, _):
                do_wave(s * SG, None)
                return 0

            lax.fori_loop(0, nsg, sg_body, 0)

            rem = wp - nsg * SG

            def shift_body(i, _):
                sel_src[pl.ds(i * 16, 16)] = sel_src[
                    pl.ds(nsg * SG + i * 16, 16)]
                sel_dst[pl.ds(i * 16, 16)] = sel_dst[
                    pl.ds(nsg * SG + i * 16, 16)]
                return 0

            @pl.when(nsg > 0)
            def _shift():
                lax.fori_loop(0, (rem + 15) // 16, shift_body, 0)

            return rem

        wp = lax.fori_loop(0, NBLK, blk_body, 0)

        for p in range(G // 16):
            sel_dst[pl.ds(wp + p * 16, 16)] = trash16
            sel_src[pl.ds(wp + p * 16, 16)] = zeros16
        do_wave(0, (wp + G - 1) // G)
        plsc.subcore_barrier()

        off = sid * 625
        pltpu.sync_copy(acc.at[pl.ds(off, 625)],
                        out_hbm.at[pl.ds(lo + off, 625)])

        plsc.subcore_barrier()


def _dinv_of(degp_ref):
    deg = jnp.sum(degp_ref[0], axis=0) + 1.0
    return lax.rsqrt(deg)


def _dense_a_body(degp_ref, x_ref, w1_ref, g1_ref):
    dinv = _dinv_of(degp_ref)
    m = jnp.dot(x_ref[...], w1_ref[...], preferred_element_type=jnp.float32)
    g1_ref[...] = m * dinv[:, None]


def _dense_b_body(degp_ref, s1_ref, g1_ref, b1_ref, w2_ref, g2_ref):
    dinv = _dinv_of(degp_ref)
    h = jnp.maximum(dinv[:, None] * (s1_ref[...] + g1_ref[...])
                    + b1_ref[...], 0.0)
    m = jnp.dot(h, w2_ref[...], preferred_element_type=jnp.float32)
    g2_ref[...] = m * dinv[:, None]


def _dense_c_body(degp_ref, s2_ref, g2_ref, b2_ref, wv_ref, bv_ref, out_ref):
    dinv = _dinv_of(degp_ref)
    h = jnp.maximum(dinv[:, None] * (s2_ref[...] + g2_ref[...])
                    + b2_ref[...], 0.0)
    out_ref[...] = jnp.dot(h, wv_ref[...],
                           preferred_element_type=jnp.float32) + bv_ref[...]


def _row_spec():
    return pl.BlockSpec((RB, D), lambda i: (i, 0))


def _deg_spec():
    return pl.BlockSpec((1, NW, RB), lambda i: (i, 0, 0))


def _full_spec(r):
    return pl.BlockSpec((r, D), lambda i: (0, 0))


_dense_a = pl.pallas_call(
    _dense_a_body,
    grid=(N // RB,),
    in_specs=[_deg_spec(), _row_spec(), _full_spec(D)],
    out_specs=_row_spec(),
    out_shape=jax.ShapeDtypeStruct((N, D), jnp.float32),
)

_dense_b = pl.pallas_call(
    _dense_b_body,
    grid=(N // RB,),
    in_specs=[_deg_spec(), _row_spec(), _row_spec(), _full_spec(1),
              _full_spec(D)],
    out_specs=_row_spec(),
    out_shape=jax.ShapeDtypeStruct((N, D), jnp.float32),
)

_dense_c = pl.pallas_call(
    _dense_c_body,
    grid=(N // RB,),
    in_specs=[_deg_spec(), _row_spec(), _row_spec(), _full_spec(1),
              _full_spec(D), _full_spec(1)],
    out_specs=_row_spec(),
    out_shape=jax.ShapeDtypeStruct((N, D), jnp.float32),
)


def kernel(x, edge_index, W1, b1, W2, b2, Wq, bq, Wk, bk, Wv, bv):
    del Wq, bq, Wk, bk
    src = edge_index[0]
    dst = edge_index[1]
    pad = EP - src.shape[0]
    src_p = jnp.concatenate([src, jnp.zeros((pad,), jnp.int32)])
    dst_p = jnp.concatenate([dst, jnp.full((pad,), N, jnp.int32)])

    degp = _deg_kernel(dst_p)
    degp = degp.reshape(NW, N // RB, RB).transpose(1, 0, 2)
    g1 = _dense_a(degp, x, W1)
    s1 = _segsum_kernel(g1, src_p, dst_p)
    g2 = _dense_b(degp, s1, g1, b1.reshape(1, D), W2)
    s2 = _segsum_kernel(g2, src_p, dst_p)
    out = _dense_c(degp, s2, g2, b2.reshape(1, D), Wv.astype(jnp.float32),
                   bv.reshape(1, D))
    return out[:, None, :]


# speedup vs baseline: 21.3941x; 1.0954x over previous
import functools

import jax
import jax.numpy as jnp
from jax import lax
from jax.experimental import pallas as pl
from jax.experimental.pallas import tpu as pltpu
from jax.experimental.pallas import tpu_sc as plsc

N = 100000
D = 64
NC = 2
NS = 16
NW = NC * NS
EPT = 40960
EP = NW * EPT
BLK = 1024
G = 64
CHUNK = 10000
NK = 5
ACC_ROWS = 10032
ZROWS = 57
RB = 4000

_mesh = plsc.VectorSubcoreMesh(
    core_axis_name="c", subcore_axis_name="s", num_cores=NC, num_subcores=NS)
_sc_params = pltpu.CompilerParams(use_tc_tiling_on_sc=False,
                                  needs_layout_passes=False)


@functools.partial(
    pl.kernel,
    out_type=jax.ShapeDtypeStruct((NW, N), jnp.float32),
    mesh=_mesh,
    compiler_params=_sc_params,
    scratch_types=[
        pltpu.VMEM((N,), jnp.float32),
        pltpu.VMEM((BLK,), jnp.int32),
    ],
)
def _deg_kernel(dst_hbm, out_hbm, deg_local, dst_blk):
    cid = lax.axis_index("c")
    sid = lax.axis_index("s")
    wid = cid * NS + sid

    zf = jnp.zeros((16,), jnp.float32)

    def zero_body(i, _):
        deg_local[pl.ds(i * 16, 16)] = zf
        return 0

    lax.fori_loop(0, N // 16, zero_body, 0)

    ones = jnp.ones((16,), jnp.float32)

    def blk_body(b, _):
        off = wid * EPT + b * BLK
        pltpu.sync_copy(dst_hbm.at[pl.ds(off, BLK)], dst_blk)

        def batch_body(i, _):
            d = dst_blk[pl.ds(i * 16, 16)]
            plsc.addupdate_scatter(deg_local, [d], ones, mask=d < N)
            return 0

        lax.fori_loop(0, BLK // 16, batch_body, 0)
        return 0

    lax.fori_loop(0, EPT // BLK, blk_body, 0)
    pltpu.sync_copy(deg_local, out_hbm.at[wid])


NWAVE = 8
SG = NWAVE * G
SELCAP = SG + BLK
NBLK = (EP // NS) // BLK


@functools.partial(
    pl.kernel,
    out_type=jax.ShapeDtypeStruct((N, D), jnp.float32),
    mesh=_mesh,
    compiler_params=_sc_params,
    scratch_types=[
        pltpu.VMEM_SHARED((ACC_ROWS, D), jnp.float32),
        pltpu.VMEM((2, BLK), jnp.int32),
        pltpu.VMEM((2, BLK), jnp.int32),
        pltpu.VMEM((SELCAP + G,), jnp.int32),
        pltpu.VMEM((SELCAP + G,), jnp.int32),
        pltpu.VMEM((NWAVE, G), jnp.int32),
        pltpu.VMEM((SG, D), jnp.float32),
        pltpu.VMEM((ZROWS, D), jnp.float32),
        pltpu.SemaphoreType.DMA,
        pltpu.SemaphoreType.DMA,
        pltpu.SemaphoreType.DMA,
    ],
)
def _segsum_kernel(g_hbm, src_hbm, dst_hbm, out_hbm,
                   acc, src_blk, dst_blk, sel_src, sel_dst,
                   stage_idx, rows, zero_buf, gsem, ssem, stg_sem):
    cid = lax.axis_index("c")
    sid = lax.axis_index("s")

    zf = jnp.zeros((16,), jnp.float32)

    def zb_body(i, _):
        zero_buf[i // 4, pl.ds((i % 4) * 16, 16)] = zf
        return 0

    lax.fori_loop(0, ZROWS * (D // 16), zb_body, 0)

    trash16 = jnp.full((16,), CHUNK, jnp.int32)
    zeros16 = jnp.zeros((16,), jnp.int32)

    def do_wave(base, ng_pred):
        gd, sd = [], []
        for g in range(NWAVE):
            def fire_g(g=g):
                gd.append(pltpu.async_copy(
                    g_hbm.at[sel_src.at[pl.ds(base + g * G, G)]],
                    rows.at[pl.ds(g * G, G)], gsem))
            if ng_pred is None:
                fire_g()
            else:
                pl.when(g < ng_pred)(fire_g)
        for g in range(NWAVE):
            if ng_pred is None:
                gd[g].wait()
            else:
                pl.when(g < ng_pred)(lambda g=g: gd[g].wait())
        for g in range(NWAVE):
            def fire_s(g=g):
                for p in range(G // 16):
                    stage_idx[g, pl.ds(p * 16, 16)] = (
                        sel_dst[pl.ds(base + g * G + p * 16, 16)])
                sd.append(pltpu.async_copy(
                    rows.at[pl.ds(g * G, G)],
                    acc.at[stage_idx.at[g]], ssem, add=True))
            if ng_pred is None:
                fire_s()
            else:
                pl.when(g < ng_pred)(fire_s)
        for g in range(len(sd)):
            if ng_pred is None:
                sd[g].wait()
            else:
                pl.when(g < ng_pred)(lambda g=g: sd[g].wait())

    for k in range(NK):
        lo = (NK * cid + k) * CHUNK

        for z in range(ACC_ROWS // NS // ZROWS):
            pltpu.sync_copy(
                zero_buf,
                acc.at[pl.ds(sid * (ACC_ROWS // NS) + z * ZROWS, ZROWS)])
        plsc.subcore_barrier()

        def stage_block(b):
            off = sid * (EP // NS) + b * BLK
            buf = lax.rem(b, 2)
            pltpu.async_copy(src_hbm.at[pl.ds(off, BLK)], src_blk.at[buf],
                             stg_sem)
            pltpu.async_copy(dst_hbm.at[pl.ds(off, BLK)], dst_blk.at[buf],
                             stg_sem)

        stage_block(0)

        def blk_body(b, wp, lo=lo):
            pltpu.make_async_copy(
                src_hbm.at[pl.ds(0, BLK)], src_blk.at[0], stg_sem).wait()
            pltpu.make_async_copy(
                src_hbm.at[pl.ds(0, BLK)], dst_blk.at[0], stg_sem).wait()
            bi = lax.rem(b, 2)

            @pl.when(b + 1 < NBLK)
            def _prefetch():
                stage_block(b + 1)

            def compact_body(i, cnt, lo=lo, bi=bi):
                d = dst_blk[bi, pl.ds(i * 16, 16)]
                sv = src_blk[bi, pl.ds(i * 16, 16)]
                m = (d >= lo) & (d < lo + CHUNK)
                plsc.store_compressed(sel_src.at[pl.ds(cnt, 16)], sv, mask=m)
                plsc.store_compressed(sel_dst.at[pl.ds(cnt, 16)], d - lo,
                                      mask=m)
                return cnt + jnp.sum(m.astype(jnp.int32))

            wp = lax.fori_loop(0, BLK // 16, compact_body, wp)

            nsg = wp // SG

            def sg_body(s, _):
                do_wave(s * SG, None)
                return 0

            lax.fori_loop(0, nsg, sg_body, 0)

            rem = wp - nsg * SG

            def shift_body(i, _):
                sel_src[pl.ds(i * 16, 16)] = sel_src[
                    pl.ds(nsg * SG + i * 16, 16)]
                sel_dst[pl.ds(i * 16, 16)] = sel_dst[
                    pl.ds(nsg * SG + i * 16, 16)]
                return 0

            @pl.when(nsg > 0)
            def _shift():
                lax.fori_loop(0, (rem + 15) // 16, shift_body, 0)

            return rem

        wp = lax.fori_loop(0, NBLK, blk_body, 0)

        for p in range(G // 16):
            sel_dst[pl.ds(wp + p * 16, 16)] = trash16
            sel_src[pl.ds(wp + p * 16, 16)] = zeros16
        do_wave(0, (wp + G - 1) // G)
        plsc.subcore_barrier()

        off = sid * 625
        pltpu.sync_copy(acc.at[pl.ds(off, 625)],
                        out_hbm.at[pl.ds(lo + off, 625)])

        plsc.subcore_barrier()


def _dinv_of(degp_ref):
    deg = jnp.sum(degp_ref[0], axis=0) + 1.0
    return lax.rsqrt(deg)


def _dense_a_body(degp_ref, x_ref, w1_ref, g1_ref):
    dinv = _dinv_of(degp_ref)
    m = jnp.dot(x_ref[...], w1_ref[...], preferred_element_type=jnp.float32)
    g1_ref[...] = m * dinv[:, None]


def _dense_b_body(degp_ref, s1_ref, g1_ref, b1_ref, w2_ref, g2_ref):
    dinv = _dinv_of(degp_ref)
    h = jnp.maximum(dinv[:, None] * (s1_ref[...] + g1_ref[...])
                    + b1_ref[...], 0.0)
    m = jnp.dot(h, w2_ref[...], preferred_element_type=jnp.float32)
    g2_ref[...] = m * dinv[:, None]


def _dense_c_body(degp_ref, s2_ref, g2_ref, b2_ref, wv_ref, bv_ref, out_ref):
    dinv = _dinv_of(degp_ref)
    h = jnp.maximum(dinv[:, None] * (s2_ref[...] + g2_ref[...])
                    + b2_ref[...], 0.0)
    out_ref[...] = jnp.dot(h, wv_ref[...],
                           preferred_element_type=jnp.float32) + bv_ref[...]


def _row_spec():
    return pl.BlockSpec((RB, D), lambda i: (i, 0))


def _deg_spec():
    return pl.BlockSpec((1, NW, RB), lambda i: (i, 0, 0))


def _full_spec(r):
    return pl.BlockSpec((r, D), lambda i: (0, 0))


_dense_a = pl.pallas_call(
    _dense_a_body,
    grid=(N // RB,),
    in_specs=[_deg_spec(), _row_spec(), _full_spec(D)],
    out_specs=_row_spec(),
    out_shape=jax.ShapeDtypeStruct((N, D), jnp.float32),
)

_dense_b = pl.pallas_call(
    _dense_b_body,
    grid=(N // RB,),
    in_specs=[_deg_spec(), _row_spec(), _row_spec(), _full_spec(1),
              _full_spec(D)],
    out_specs=_row_spec(),
    out_shape=jax.ShapeDtypeStruct((N, D), jnp.float32),
)

_dense_c = pl.pallas_call(
    _dense_c_body,
    grid=(N // RB,),
    in_specs=[_deg_spec(), _row_spec(), _row_spec(), _full_spec(1),
              _full_spec(D), _full_spec(1)],
    out_specs=_row_spec(),
    out_shape=jax.ShapeDtypeStruct((N, D), jnp.float32),
)


def kernel(x, edge_index, W1, b1, W2, b2, Wq, bq, Wk, bk, Wv, bv):
    del Wq, bq, Wk, bk
    src = edge_index[0]
    dst = edge_index[1]
    pad = EP - src.shape[0]
    src_p = jnp.concatenate([src, jnp.zeros((pad,), jnp.int32)])
    dst_p = jnp.concatenate([dst, jnp.full((pad,), N, jnp.int32)])

    degp = _deg_kernel(dst_p)
    degp = degp.reshape(NW, N // RB, RB).transpose(1, 0, 2)
    g1 = _dense_a(degp, x, W1)
    s1 = _segsum_kernel(g1, src_p, dst_p)
    g2 = _dense_b(degp, s1, g1, b1.reshape(1, D), W2)
    s2 = _segsum_kernel(g2, src_p, dst_p)
    out = _dense_c(degp, s2, g2, b2.reshape(1, D), Wv.astype(jnp.float32),
                   bv.reshape(1, D))
    return out[:, None, :]
